# baseline (speedup 1.0000x reference)
import math

import jax
import jax.numpy as jnp
import numpy as np
from jax.experimental import pallas as pl
from jax.experimental.pallas import tpu as pltpu

N_GRID = 48
GRID = 0.5
SHIFT = N_GRID * 0.5 - 0.5
XW = 11
N_ATOMS = 1024
_E2 = math.exp(2.0)
_LOG2E = math.log2(math.e)

NGRP = 3
YG = 16
SUB = 8
LANE = 96
DIM0 = N_GRID * NGRP

T_BRANCH = np.float32(-2.0 * _LOG2E)
T_EDGE = np.float32(-4.5 * _LOG2E)

DEG = 3

N_ACC = 4


def _cheb_nodes(a, b, n):
    k = np.arange(n)
    x = np.cos((2 * k + 1) * np.pi / (2 * n))
    return 0.5 * (a + b) + 0.5 * (b - a) * x


def _fit_channel(r):
    r2 = r * r
    T = _cheb_nodes(float(T_EDGE) + 1e-6, float(T_BRANCH), 512)
    u = T * r2 / (-2.0 * _LOG2E)
    d = np.sqrt(u)
    f2 = (4.0 / (_E2 * r2)) * (d - 1.5 * r) ** 2
    q = f2 / (T - float(T_EDGE))
    c = np.polynomial.chebyshev.chebfit(T, q, DEG)
    return [float(v) for v in np.polynomial.chebyshev.cheb2poly(c)]


_RADII = (1.7, 1.55, 1.52)
_POLYS = [_fit_channel(r) for r in _RADII]


def _sel3(ch, a, b, c):
    return jnp.where(ch == 0, a, jnp.where(ch == 1, b, c)).astype(jnp.float32)


def _splat_kernel(vecs_ref, out_ref, acc_ref):
    ch = pl.program_id(0)

    s = jax.lax.broadcasted_iota(jnp.int32, (SUB, LANE), 0)
    c = jax.lax.broadcasted_iota(jnp.int32, (SUB, LANE), 1)
    flat = s * LANE + c
    ylc = (flat // N_GRID).astype(jnp.float32) * GRID
    zc = (flat % N_GRID).astype(jnp.float32) * GRID

    r = _sel3(ch, *_RADII)
    h = 3.0 * r
    sc = jnp.float32(-2.0 * _LOG2E) / (r * r)
    q = [_sel3(ch, _POLYS[0][k], _POLYS[1][k], _POLYS[2][k])
         for k in range(DEG + 1)]

    acc_ref[...] = jnp.zeros_like(acc_ref)

    def one_atom(i, k):
        vx = vecs_ref[0, 0, 3 * i]
        vy = vecs_ref[0, 0, 3 * i + 1]
        vz = vecs_ref[0, 0, 3 * i + 2]
        x0 = jnp.clip(jnp.floor(2.0 * vx - h).astype(jnp.int32) + 1, 0,
                      N_GRID - XW)
        y0 = jnp.clip(jnp.floor(2.0 * vy - h).astype(jnp.int32) + 1, 0,
                      N_GRID - XW)
        g0 = y0 // YG
        valid2 = (g0 < NGRP - 1).astype(jnp.float32)
        dzv = vz - zc
        dz2 = dzv * dzv
        dy0 = (vy - 8.0 * g0.astype(jnp.float32)) - ylc
        dy1 = dy0 - 8.0
        plane0 = (dy0 * dy0 + dz2) * sc
        plane1 = (dy1 * dy1 + dz2) * sc
        x0f = x0.astype(jnp.float32) * GRID
        base0 = (x0 * NGRP + g0).astype(jnp.int32)

        def row(T, idx):
            Tc = jnp.maximum(T, T_EDGE)
            g1 = jnp.exp2(T)
            qv = (q[3] * Tc + q[2]) * (Tc * Tc) + (q[1] * Tc + q[0])
            g2 = (Tc - T_EDGE) * qv
            return jnp.where(T > T_BRANCH, g1, g2)

        for j in range(XW):
            dxj = vx - (x0f + j * GRID)
            dxT = dxj * dxj * sc
            base = base0 + j * NGRP
            m0 = row(dxT + plane0, base)
            acc_ref[k, base, :, :] += m0
            m1 = row(dxT + plane1, base + 1) * valid2
            idx1 = jnp.minimum(base + 1, DIM0 - 1)
            acc_ref[k, idx1, :, :] += m1

    def body(i, _):
        for k in range(N_ACC):
            one_atom(N_ACC * i + k, k)
        return 0

    jax.lax.fori_loop(0, N_ATOMS // N_ACC, body, 0)
    total = acc_ref[0]
    for k in range(1, N_ACC):
        total = total + acc_ref[k]
    out_ref[0] = total


@jax.jit
def kernel(vecs_C, vecs_N, vecs_O):
    vecs = (jnp.stack([vecs_C, vecs_N, vecs_O], axis=0)
            + SHIFT).reshape(3, 1, 3 * N_ATOMS)
    out = pl.pallas_call(
        _splat_kernel,
        grid=(3,),
        in_specs=[
            pl.BlockSpec((1, 1, 3 * N_ATOMS), lambda ch: (ch, 0, 0),
                         memory_space=pltpu.SMEM),
        ],
        out_specs=pl.BlockSpec((1, DIM0, SUB, LANE),
                               lambda ch: (ch, 0, 0, 0)),
        out_shape=jax.ShapeDtypeStruct((3, DIM0, SUB, LANE), jnp.float32),
        scratch_shapes=[pltpu.VMEM((N_ACC, DIM0, SUB, LANE), jnp.float32)],
    )(vecs)
    return out.reshape(3, N_GRID, NGRP * YG, N_GRID)


# speedup vs baseline: 1.3940x; 1.3940x over previous
import math

import jax
import jax.numpy as jnp
import numpy as np
from jax.experimental import pallas as pl
from jax.experimental.pallas import tpu as pltpu

N_GRID = 48
GRID = 0.5
SHIFT = N_GRID * 0.5 - 0.5
XW = 11
N_ATOMS = 1024
_E2 = math.exp(2.0)

SUB = 8
LANE = 288

DEG = 3

N_ACC = 4


def _cheb_nodes(a, b, n):
    k = np.arange(n)
    x = np.cos((2 * k + 1) * np.pi / (2 * n))
    return 0.5 * (a + b) + 0.5 * (b - a) * x


def _fit_channel(r):
    r2 = r * r
    t = _cheb_nodes(-4.5, -2.0, 512)
    d = np.sqrt(t * r2 / -2.0)
    cc = 4.0 / (_E2 * r2)
    qt = (r2 / 2.0) * cc * (1.5 * r - d) / (1.5 * r + d)
    c = np.polynomial.chebyshev.chebfit(t, qt, DEG)
    return [float(v) for v in np.polynomial.chebyshev.cheb2poly(c)]


_RADII = (1.7, 1.55, 1.52)
_POLYS = [_fit_channel(r) for r in _RADII]


def _sel3(ch, a, b, c):
    return jnp.where(ch == 0, a, jnp.where(ch == 1, b, c)).astype(jnp.float32)


def _splat_kernel(vecs_ref, out_ref, acc_ref):
    ch = pl.program_id(0)

    s = jax.lax.broadcasted_iota(jnp.int32, (SUB, LANE), 0)
    c = jax.lax.broadcasted_iota(jnp.int32, (SUB, LANE), 1)
    flat = s * LANE + c
    ycoord = (flat // N_GRID).astype(jnp.float32) * GRID
    zcoord = (flat % N_GRID).astype(jnp.float32) * GRID

    r = _sel3(ch, *_RADII)
    h = 3.0 * r
    inv_r2 = -2.0 / (r * r)
    q = [_sel3(ch, _POLYS[0][k], _POLYS[1][k], _POLYS[2][k])
         for k in range(DEG + 1)]

    acc_ref[...] = jnp.zeros_like(acc_ref)

    def one_atom(i, k):
        vx = vecs_ref[0, 0, 3 * i]
        vy = vecs_ref[0, 0, 3 * i + 1]
        vz = vecs_ref[0, 0, 3 * i + 2]
        x0 = jnp.clip(jnp.floor(2.0 * vx - h).astype(jnp.int32) + 1, 0,
                      N_GRID - XW)
        dyz2t = ((vy - ycoord) ** 2 + (vz - zcoord) ** 2) * inv_r2
        x0f = x0.astype(jnp.float32) * GRID
        for j in range(XW):
            dxj = vx - (x0f + j * GRID)
            t = dxj * dxj * inv_r2 + dyz2t
            tc = jnp.maximum(t, -4.5)
            g1 = jnp.exp(t)
            qv = (q[3] * tc + q[2]) * (tc * tc) + (q[1] * tc + q[0])
            g2 = (tc + 4.5) * qv
            m = jnp.where(t > -2.0, g1, g2)
            acc_ref[k, x0 + j, :, :] += m

    def body(i, _):
        for k in range(N_ACC):
            one_atom(N_ACC * i + k, k)
        return 0

    jax.lax.fori_loop(0, N_ATOMS // N_ACC, body, 0)
    total = acc_ref[0]
    for k in range(1, N_ACC):
        total = total + acc_ref[k]
    out_ref[0] = total


@jax.jit
def kernel(vecs_C, vecs_N, vecs_O):
    vecs = (jnp.stack([vecs_C, vecs_N, vecs_O], axis=0)
            + SHIFT).reshape(3, 1, 3 * N_ATOMS)
    out = pl.pallas_call(
        _splat_kernel,
        grid=(3,),
        in_specs=[
            pl.BlockSpec((1, 1, 3 * N_ATOMS), lambda ch: (ch, 0, 0),
                         memory_space=pltpu.SMEM),
        ],
        out_specs=pl.BlockSpec((1, N_GRID, SUB, LANE),
                               lambda ch: (ch, 0, 0, 0)),
        out_shape=jax.ShapeDtypeStruct((3, N_GRID, SUB, LANE), jnp.float32),
        scratch_shapes=[pltpu.VMEM((N_ACC, N_GRID, SUB, LANE), jnp.float32)],
    )(vecs)
    return out.reshape(3, N_GRID, N_GRID, N_GRID)


# speedup vs baseline: 1.8471x; 1.3250x over previous
import math

import jax
import jax.numpy as jnp
import numpy as np
from jax.experimental import pallas as pl
from jax.experimental.pallas import tpu as pltpu

N_GRID = 48
GRID = 0.5
SHIFT = N_GRID * 0.5 - 0.5
XW = 11
N_ATOMS = 1024
_E2 = math.exp(2.0)

NGRP = 3
YG = 16
ZPAD = 64
SUB = 8
LANE = 128
DIM0 = N_GRID * NGRP

DEG = 3

N_ACC = 4


def _cheb_nodes(a, b, n):
    k = np.arange(n)
    x = np.cos((2 * k + 1) * np.pi / (2 * n))
    return 0.5 * (a + b) + 0.5 * (b - a) * x


def _fit_channel(r):
    r2 = r * r
    t = _cheb_nodes(-4.5, -2.0, 512)
    d = np.sqrt(t * r2 / -2.0)
    cc = 4.0 / (_E2 * r2)
    qt = (r2 / 2.0) * cc * (1.5 * r - d) / (1.5 * r + d)
    c = np.polynomial.chebyshev.chebfit(t, qt, DEG)
    return [float(v) for v in np.polynomial.chebyshev.cheb2poly(c)]


_RADII = (1.7, 1.55, 1.52)
_POLYS = [_fit_channel(r) for r in _RADII]


def _sel3(ch, a, b, c):
    return jnp.where(ch == 0, a, jnp.where(ch == 1, b, c)).astype(jnp.float32)


def _splat_kernel(vecs_ref, out_ref, acc_ref):
    ch = pl.program_id(0)

    s = jax.lax.broadcasted_iota(jnp.int32, (SUB, LANE), 0)
    c = jax.lax.broadcasted_iota(jnp.int32, (SUB, LANE), 1)
    flat = s * LANE + c
    ycoord = (flat // ZPAD).astype(jnp.float32) * GRID
    zcoord = (flat % ZPAD).astype(jnp.float32) * GRID

    r = _sel3(ch, *_RADII)
    h = 3.0 * r
    inv_r2 = -2.0 / (r * r)
    q = [_sel3(ch, _POLYS[0][k], _POLYS[1][k], _POLYS[2][k])
         for k in range(DEG + 1)]

    acc_ref[...] = jnp.zeros_like(acc_ref)

    def one_atom(i, k):
        vx = vecs_ref[0, 0, 3 * i]
        vy = vecs_ref[0, 0, 3 * i + 1]
        vz = vecs_ref[0, 0, 3 * i + 2]
        x0 = jnp.clip(jnp.floor(2.0 * vx - h).astype(jnp.int32) + 1, 0,
                      N_GRID - XW)
        y0 = jnp.clip(jnp.floor(2.0 * vy - h).astype(jnp.int32) + 1, 0,
                      N_GRID - XW)
        g0 = jnp.minimum(y0 // YG, NGRP - 2)
        dzv = vz - zcoord
        dz2 = dzv * dzv
        dy0 = (vy - 8.0 * g0.astype(jnp.float32)) - ycoord
        dy1 = dy0 - 8.0
        plane0 = (dy0 * dy0 + dz2) * inv_r2
        plane1 = (dy1 * dy1 + dz2) * inv_r2
        x0f = x0.astype(jnp.float32) * GRID
        base0 = x0 * NGRP + g0

        def fval(t):
            tc = jnp.maximum(t, -4.5)
            g1 = jnp.exp(t)
            qv = (q[3] * tc + q[2]) * (tc * tc) + (q[1] * tc + q[0])
            g2 = (tc + 4.5) * qv
            return jnp.where(t > -2.0, g1, g2)

        for j in range(XW):
            dxj = vx - (x0f + j * GRID)
            dxt = dxj * dxj * inv_r2
            m0 = fval(dxt + plane0)
            m1 = fval(dxt + plane1)
            m = jnp.stack([m0, m1], axis=0)
            acc_ref[k, pl.ds(base0 + j * NGRP, 2), :, :] += m

    def body(i, _):
        for k in range(N_ACC):
            one_atom(N_ACC * i + k, k)
        return 0

    jax.lax.fori_loop(0, N_ATOMS // N_ACC, body, 0)
    total = acc_ref[0]
    for k in range(1, N_ACC):
        total = total + acc_ref[k]
    out_ref[0] = total


@jax.jit
def kernel(vecs_C, vecs_N, vecs_O):
    vecs = (jnp.stack([vecs_C, vecs_N, vecs_O], axis=0)
            + SHIFT).reshape(3, 1, 3 * N_ATOMS)
    out = pl.pallas_call(
        _splat_kernel,
        grid=(3,),
        in_specs=[
            pl.BlockSpec((1, 1, 3 * N_ATOMS), lambda ch: (ch, 0, 0),
                         memory_space=pltpu.SMEM),
        ],
        out_specs=pl.BlockSpec((1, DIM0, SUB, LANE),
                               lambda ch: (ch, 0, 0, 0)),
        out_shape=jax.ShapeDtypeStruct((3, DIM0, SUB, LANE), jnp.float32),
        scratch_shapes=[pltpu.VMEM((N_ACC, DIM0, SUB, LANE), jnp.float32)],
    )(vecs)
    out = out.reshape(3, N_GRID, NGRP, YG, ZPAD)[..., :N_GRID]
    return out.reshape(3, N_GRID, NGRP * YG, N_GRID)


# speedup vs baseline: 1.9286x; 1.0441x over previous
import math

import jax
import jax.numpy as jnp
import numpy as np
from jax.experimental import pallas as pl
from jax.experimental.pallas import tpu as pltpu

N_GRID = 48
GRID = 0.5
SHIFT = N_GRID * 0.5 - 0.5
XW = 11
N_ATOMS = 1024
_E2 = math.exp(2.0)
_LOG2E = math.log2(math.e)
_T_BRANCH = -2.0 * _LOG2E
_T_EDGE = float(__import__('numpy').float32(-4.5 * _LOG2E))

NGRP = 3
YG = 16
ZPAD = 64
SUB = 8
LANE = 128
DIM0 = N_GRID * NGRP

DEG = 3

N_ACC = 4


def _cheb_nodes(a, b, n):
    k = np.arange(n)
    x = np.cos((2 * k + 1) * np.pi / (2 * n))
    return 0.5 * (a + b) + 0.5 * (b - a) * x


def _fit_channel(r):
    r2 = r * r
    T = _cheb_nodes(_T_EDGE + 1e-6, _T_BRANCH, 512)
    u = T * r2 / (-2.0 * _LOG2E)
    d = np.sqrt(u)
    f2 = (4.0 / (_E2 * r2)) * (d - 1.5 * r) ** 2
    qt = f2 / (T - _T_EDGE)
    c = np.polynomial.chebyshev.chebfit(T, qt, DEG)
    return [float(v) for v in np.polynomial.chebyshev.cheb2poly(c)]


_RADII = (1.7, 1.55, 1.52)
_POLYS = [_fit_channel(r) for r in _RADII]


def _sel3(ch, a, b, c):
    return jnp.where(ch == 0, a, jnp.where(ch == 1, b, c)).astype(jnp.float32)


def _splat_kernel(vecs_ref, out_ref, acc_ref):
    ch = pl.program_id(0)

    s = jax.lax.broadcasted_iota(jnp.int32, (SUB, LANE), 0)
    c = jax.lax.broadcasted_iota(jnp.int32, (SUB, LANE), 1)
    flat = s * LANE + c
    ycoord = (flat // ZPAD).astype(jnp.float32) * GRID
    zcoord = (flat % ZPAD).astype(jnp.float32) * GRID

    r = _sel3(ch, *_RADII)
    h = 3.0 * r
    inv_r2 = jnp.float32(-2.0 * _LOG2E) / (r * r)
    q = [_sel3(ch, _POLYS[0][k], _POLYS[1][k], _POLYS[2][k])
         for k in range(DEG + 1)]

    acc_ref[...] = jnp.zeros_like(acc_ref)

    def one_atom(i, k):
        vx = vecs_ref[0, 0, 3 * i]
        vy = vecs_ref[0, 0, 3 * i + 1]
        vz = vecs_ref[0, 0, 3 * i + 2]
        x0 = jnp.clip(jnp.floor(2.0 * vx - h).astype(jnp.int32) + 1, 0,
                      N_GRID - XW)
        y0 = jnp.clip(jnp.floor(2.0 * vy - h).astype(jnp.int32) + 1, 0,
                      N_GRID - XW)
        g0 = jnp.minimum(y0 // YG, NGRP - 2)
        dzv = vz - zcoord
        dz2 = dzv * dzv
        dy0 = (vy - 8.0 * g0.astype(jnp.float32)) - ycoord
        dy1 = dy0 - 8.0
        plane0 = (dy0 * dy0 + dz2) * inv_r2
        plane1 = (dy1 * dy1 + dz2) * inv_r2
        x0f = x0.astype(jnp.float32) * GRID
        base0 = x0 * NGRP + g0

        def fval(t):
            tc = jnp.maximum(t, _T_EDGE)
            g1 = jnp.exp2(t)
            qv = (q[3] * tc + q[2]) * (tc * tc) + (q[1] * tc + q[0])
            g2 = (tc - _T_EDGE) * qv
            return jnp.where(t > _T_BRANCH, g1, g2)

        for j in range(XW):
            dxj = vx - (x0f + j * GRID)
            dxt = dxj * dxj * inv_r2
            m0 = fval(dxt + plane0)
            m1 = fval(dxt + plane1)
            m = jnp.stack([m0, m1], axis=0)
            acc_ref[k, pl.ds(base0 + j * NGRP, 2), :, :] += m

    def body(i, _):
        for k in range(N_ACC):
            one_atom(N_ACC * i + k, k)
        return 0

    jax.lax.fori_loop(0, N_ATOMS // N_ACC, body, 0)
    total = acc_ref[0]
    for k in range(1, N_ACC):
        total = total + acc_ref[k]
    out_ref[0] = total


@jax.jit
def kernel(vecs_C, vecs_N, vecs_O):
    vecs = (jnp.stack([vecs_C, vecs_N, vecs_O], axis=0)
            + SHIFT).reshape(3, 1, 3 * N_ATOMS)
    out = pl.pallas_call(
        _splat_kernel,
        grid=(3,),
        in_specs=[
            pl.BlockSpec((1, 1, 3 * N_ATOMS), lambda ch: (ch, 0, 0),
                         memory_space=pltpu.SMEM),
        ],
        out_specs=pl.BlockSpec((1, DIM0, SUB, LANE),
                               lambda ch: (ch, 0, 0, 0)),
        out_shape=jax.ShapeDtypeStruct((3, DIM0, SUB, LANE), jnp.float32),
        scratch_shapes=[pltpu.VMEM((N_ACC, DIM0, SUB, LANE), jnp.float32)],
    )(vecs)
    out = out.reshape(3, N_GRID, NGRP, YG, ZPAD)[..., :N_GRID]
    return out.reshape(3, N_GRID, NGRP * YG, N_GRID)
